# baseline (speedup 1.0000x reference)
import functools

import jax
import jax.numpy as jnp
from jax import lax
from jax.experimental import pallas as pl
from jax.experimental.pallas import tpu as pltpu
from jax.experimental.pallas import tpu_sc as plsc

_NC = 2
_NS = 16
_NW = _NC * _NS


def _route_body(x_ref, wr_ref, slot_ref, xg_ref, carry_ref, *, E, C, R):
    b = pl.program_id(0)

    @pl.when(b == 0)
    def _init():
        carry_ref[...] = jnp.zeros_like(carry_ref)

    x = x_ref[...]
    logits = jnp.dot(x, wr_ref[...], preferred_element_type=jnp.float32)
    m = jnp.max(logits, axis=1, keepdims=True)
    col = lax.broadcasted_iota(jnp.int32, logits.shape, 1)
    e_idx = jnp.min(jnp.where(logits == m, col, E), axis=1, keepdims=True)
    gate = 1.0 / jnp.sum(jnp.exp(logits - m), axis=1, keepdims=True)

    onehot = (col == e_idx).astype(jnp.float32)
    cum = onehot
    k = 1
    while k < R:
        shifted = jnp.concatenate([jnp.zeros((k, E), jnp.float32), cum[:-k]], 0)
        cum = cum + shifted
        k *= 2
    total = cum + carry_ref[...]
    carry_ref[...] = total[-1:, :]
    pos = jnp.sum((total - 1.0) * onehot, axis=1, keepdims=True)
    pos_i = pos.astype(jnp.int32)
    keep = pos_i < C
    slot_ref[...] = jnp.where(keep, e_idx * C + pos_i, E * C)
    xg_ref[...] = x * jnp.where(keep, gate, 0.0)


def _route(x2d, w_router, E, C, R):
    T, d = x2d.shape
    grid = (T // R,)
    slot2d, xg = pl.pallas_call(
        functools.partial(_route_body, E=E, C=C, R=R),
        grid=grid,
        in_specs=[
            pl.BlockSpec((R, d), lambda b: (b, 0)),
            pl.BlockSpec((d, E), lambda b: (0, 0)),
        ],
        out_specs=[
            pl.BlockSpec((R, 1), lambda b: (b, 0)),
            pl.BlockSpec((R, d), lambda b: (b, 0)),
        ],
        out_shape=[
            jax.ShapeDtypeStruct((T, 1), jnp.int32),
            jax.ShapeDtypeStruct((T, d), jnp.float32),
        ],
        scratch_shapes=[pltpu.VMEM((1, E), jnp.float32)],
    )(x2d, w_router)
    return slot2d.reshape(T), xg


def _dispatch(xg, slot3d, n_rows, CH):
    T, d = xg.shape
    tpw = T // _NW
    nch = tpw // CH
    mesh = plsc.VectorSubcoreMesh(core_axis_name="c", subcore_axis_name="s")

    @functools.partial(
        pl.kernel,
        out_type=jax.ShapeDtypeStruct((n_rows, d), jnp.float32),
        mesh=mesh,
        scratch_types=[
            pltpu.VMEM((nch, CH), jnp.int32),
            pltpu.VMEM((2, CH, d), jnp.float32),
            pltpu.SemaphoreType.DMA,
            pltpu.SemaphoreType.DMA,
            pltpu.SemaphoreType.DMA,
            pltpu.SemaphoreType.DMA,
        ],
    )
    def run(xg_hbm, slot_hbm, buf_hbm, idx_v, rows_v, ls0, ls1, ss0, ss1):
        del ls1, ss1
        wid = lax.axis_index("s") * _NC + lax.axis_index("c")
        base = wid * tpw
        pltpu.sync_copy(slot_hbm.at[wid], idx_v)
        sc = {}
        for k in range(nch):
            s = k & 1
            pltpu.sync_copy(xg_hbm.at[pl.ds(base + k * CH, CH)], rows_v.at[s])
            if k >= 1:
                sc[k - 1].wait()
            sc[k] = pltpu.async_copy(
                rows_v.at[s], buf_hbm.at[idx_v.at[k]], ss0)
        sc[nch - 1].wait()

    return run(xg, slot3d)


def _combine(o, slot3d, T, CH):
    n_rows, d = o.shape
    tpw = T // _NW
    nch = tpw // CH
    mesh = plsc.VectorSubcoreMesh(core_axis_name="c", subcore_axis_name="s")

    @functools.partial(
        pl.kernel,
        out_type=jax.ShapeDtypeStruct((T, d), jnp.float32),
        mesh=mesh,
        scratch_types=[
            pltpu.VMEM((nch, CH), jnp.int32),
            pltpu.VMEM((2, CH, d), jnp.float32),
            pltpu.SemaphoreType.DMA,
            pltpu.SemaphoreType.DMA,
            pltpu.SemaphoreType.DMA,
            pltpu.SemaphoreType.DMA,
        ],
    )
    def run(o_hbm, slot_hbm, y_hbm, idx_v, rows_v, gs0, gs1, ws0, ws1):
        del gs1, ws1
        wid = lax.axis_index("s") * _NC + lax.axis_index("c")
        base = wid * tpw
        pltpu.sync_copy(slot_hbm.at[wid], idx_v)
        w = {}
        for k in range(nch):
            s = k & 1
            pltpu.async_copy(o_hbm.at[idx_v.at[k]], rows_v.at[s], gs0).wait()
            if k >= 1:
                w[k - 1].wait()
            w[k] = pltpu.async_copy(
                rows_v.at[s], y_hbm.at[pl.ds(base + k * CH, CH)], ws0)
        w[nch - 1].wait()

    return run(o, slot3d)


def _ffn_body(buf_ref, w1_ref, w2_ref, o_ref, *, E):
    e = pl.program_id(0)

    @pl.when(e < E)
    def _compute():
        h = jnp.maximum(
            jnp.dot(buf_ref[...], w1_ref[0],
                    preferred_element_type=jnp.float32),
            0.0,
        )
        o_ref[...] = jnp.dot(h, w2_ref[0], preferred_element_type=jnp.float32)

    @pl.when(e == E)
    def _sentinel():
        o_ref[...] = jnp.zeros_like(o_ref)


def _ffn(buf, w1, w2, E, C):
    d = buf.shape[1]
    f = w1.shape[2]
    return pl.pallas_call(
        functools.partial(_ffn_body, E=E),
        grid=(E + 1,),
        in_specs=[
            pl.BlockSpec((C, d), lambda e: (e, 0)),
            pl.BlockSpec((1, d, f), lambda e: (jnp.minimum(e, E - 1), 0, 0)),
            pl.BlockSpec((1, f, d), lambda e: (jnp.minimum(e, E - 1), 0, 0)),
        ],
        out_specs=pl.BlockSpec((C, d), lambda e: (e, 0)),
        out_shape=jax.ShapeDtypeStruct(((E + 1) * C, d), jnp.float32),
    )(buf, w1, w2)


def kernel(inputs, w_router, w1, w2):
    Bq, Sq, d = inputs.shape
    T = Bq * Sq
    E = w1.shape[0]
    C = int(1.25 * T / E)

    x2d = inputs.reshape(T, d)
    CH = 64
    slot, xg = _route(x2d, w_router, E, C, R=512)
    slot3d = slot.reshape(_NW, T // _NW // CH, CH)
    buf = _dispatch(xg, slot3d, n_rows=E * C + 8, CH=CH)
    o = _ffn(buf, w1, w2, E, C)
    y = _combine(o, slot3d, T, CH=CH)
    return y.reshape(Bq, Sq, d)


# speedup vs baseline: 1.0015x; 1.0015x over previous
import functools

import jax
import jax.numpy as jnp
from jax import lax
from jax.experimental import pallas as pl
from jax.experimental.pallas import tpu as pltpu
from jax.experimental.pallas import tpu_sc as plsc

_NC = 2
_NS = 16
_NW = _NC * _NS


def _route_body(x_ref, wr_ref, slot_ref, xg_ref, carry_ref, *, E, C, R):
    b = pl.program_id(0)

    @pl.when(b == 0)
    def _init():
        carry_ref[...] = jnp.zeros_like(carry_ref)

    x = x_ref[...]
    logits = jnp.dot(x, wr_ref[...], preferred_element_type=jnp.float32)
    m = jnp.max(logits, axis=1, keepdims=True)
    col = lax.broadcasted_iota(jnp.int32, logits.shape, 1)
    e_idx = jnp.min(jnp.where(logits == m, col, E), axis=1, keepdims=True)
    gate = 1.0 / jnp.sum(jnp.exp(logits - m), axis=1, keepdims=True)

    onehot = (col == e_idx).astype(jnp.float32)
    cum = onehot
    k = 1
    while k < R:
        shifted = jnp.concatenate([jnp.zeros((k, E), jnp.float32), cum[:-k]], 0)
        cum = cum + shifted
        k *= 2
    total = cum + carry_ref[...]
    carry_ref[...] = total[-1:, :]
    pos = jnp.sum((total - 1.0) * onehot, axis=1, keepdims=True)
    pos_i = pos.astype(jnp.int32)
    keep = pos_i < C
    slot_ref[...] = jnp.where(keep, e_idx * C + pos_i, E * C)
    xg_ref[...] = x * jnp.where(keep, gate, 0.0)


def _route(x2d, w_router, E, C, R):
    T, d = x2d.shape
    grid = (T // R,)
    slot2d, xg = pl.pallas_call(
        functools.partial(_route_body, E=E, C=C, R=R),
        grid=grid,
        in_specs=[
            pl.BlockSpec((R, d), lambda b: (b, 0)),
            pl.BlockSpec((d, E), lambda b: (0, 0)),
        ],
        out_specs=[
            pl.BlockSpec((R, 1), lambda b: (b, 0)),
            pl.BlockSpec((R, d), lambda b: (b, 0)),
        ],
        out_shape=[
            jax.ShapeDtypeStruct((T, 1), jnp.int32),
            jax.ShapeDtypeStruct((T, d), jnp.float32),
        ],
        scratch_shapes=[pltpu.VMEM((1, E), jnp.float32)],
    )(x2d, w_router)
    return slot2d.reshape(T), xg


def _dispatch(xg, slot3d, n_rows, CH):
    T, d = xg.shape
    tpw = T // _NW
    nch = tpw // CH
    mesh = plsc.VectorSubcoreMesh(core_axis_name="c", subcore_axis_name="s")

    @functools.partial(
        pl.kernel,
        out_type=jax.ShapeDtypeStruct((n_rows, d), jnp.float32),
        mesh=mesh,
        scratch_types=[
            pltpu.VMEM((nch, CH), jnp.int32),
            pltpu.VMEM((2, CH, d), jnp.float32),
            pltpu.SemaphoreType.DMA,
            pltpu.SemaphoreType.DMA,
            pltpu.SemaphoreType.DMA,
            pltpu.SemaphoreType.DMA,
        ],
    )
    def run(xg_hbm, slot_hbm, buf_hbm, idx_v, rows_v, ls0, ls1, ss0, ss1):
        del ls1, ss1
        wid = lax.axis_index("s") * _NC + lax.axis_index("c")
        base = wid * tpw
        pltpu.sync_copy(slot_hbm.at[wid], idx_v)
        sc = {}
        for k in range(nch):
            s = k & 1
            pltpu.sync_copy(xg_hbm.at[pl.ds(base + k * CH, CH)], rows_v.at[s])
            if k >= 1:
                sc[k - 1].wait()
            sc[k] = pltpu.async_copy(
                rows_v.at[s], buf_hbm.at[idx_v.at[k]], ss0)
        sc[nch - 1].wait()

    return run(xg, slot3d)


def _combine(o, slot3d, T, CH):
    n_rows, d = o.shape
    tpw = T // _NW
    nch = tpw // CH
    mesh = plsc.VectorSubcoreMesh(core_axis_name="c", subcore_axis_name="s")

    @functools.partial(
        pl.kernel,
        out_type=jax.ShapeDtypeStruct((T, d), jnp.float32),
        mesh=mesh,
        scratch_types=[
            pltpu.VMEM((nch, CH), jnp.int32),
            pltpu.VMEM((2, CH, d), jnp.float32),
            pltpu.SemaphoreType.DMA,
            pltpu.SemaphoreType.DMA,
            pltpu.SemaphoreType.DMA,
            pltpu.SemaphoreType.DMA,
        ],
    )
    def run(o_hbm, slot_hbm, y_hbm, idx_v, rows_v, gs0, gs1, ws0, ws1):
        del gs1, ws1
        wid = lax.axis_index("s") * _NC + lax.axis_index("c")
        base = wid * tpw
        pltpu.sync_copy(slot_hbm.at[wid], idx_v)
        w = {}
        for k in range(nch):
            s = k & 1
            pltpu.async_copy(o_hbm.at[idx_v.at[k]], rows_v.at[s], gs0).wait()
            if k >= 1:
                w[k - 1].wait()
            w[k] = pltpu.async_copy(
                rows_v.at[s], y_hbm.at[pl.ds(base + k * CH, CH)], ws0)
        w[nch - 1].wait()

    return run(o, slot3d)


def _ffn_body(buf_ref, w1_ref, w2_ref, o_ref, *, E):
    e = pl.program_id(0)

    @pl.when(e < E)
    def _compute():
        x = buf_ref[...].astype(jnp.bfloat16)
        h = jnp.maximum(
            jnp.dot(x, w1_ref[0].astype(jnp.bfloat16),
                    preferred_element_type=jnp.float32),
            0.0,
        )
        o_ref[...] = jnp.dot(h.astype(jnp.bfloat16),
                             w2_ref[0].astype(jnp.bfloat16),
                             preferred_element_type=jnp.float32)

    @pl.when(e == E)
    def _sentinel():
        o_ref[...] = jnp.zeros_like(o_ref)


def _ffn(buf, w1, w2, E, C):
    d = buf.shape[1]
    f = w1.shape[2]
    return pl.pallas_call(
        functools.partial(_ffn_body, E=E),
        grid=(E + 1,),
        in_specs=[
            pl.BlockSpec((C, d), lambda e: (e, 0)),
            pl.BlockSpec((1, d, f), lambda e: (jnp.minimum(e, E - 1), 0, 0)),
            pl.BlockSpec((1, f, d), lambda e: (jnp.minimum(e, E - 1), 0, 0)),
        ],
        out_specs=pl.BlockSpec((C, d), lambda e: (e, 0)),
        out_shape=jax.ShapeDtypeStruct(((E + 1) * C, d), jnp.float32),
    )(buf, w1, w2)


def kernel(inputs, w_router, w1, w2):
    Bq, Sq, d = inputs.shape
    T = Bq * Sq
    E = w1.shape[0]
    C = int(1.25 * T / E)

    x2d = inputs.reshape(T, d)
    CH = 64
    slot, xg = _route(x2d, w_router, E, C, R=512)
    slot3d = slot.reshape(_NW, T // _NW // CH, CH)
    buf = _dispatch(xg, slot3d, n_rows=E * C + 8, CH=CH)
    o = _ffn(buf, w1, w2, E, C)
    y = _combine(o, slot3d, T, CH=CH)
    return y.reshape(Bq, Sq, d)


# speedup vs baseline: 1.0315x; 1.0299x over previous
import functools

import jax
import jax.numpy as jnp
from jax import lax
from jax.experimental import pallas as pl
from jax.experimental.pallas import tpu as pltpu
from jax.experimental.pallas import tpu_sc as plsc

_NC = 2
_NS = 16
_NW = _NC * _NS


def _route_body(x_ref, wr_ref, slot_ref, xg_ref, carry_ref, *, E, C, R):
    b = pl.program_id(0)

    @pl.when(b == 0)
    def _init():
        carry_ref[...] = jnp.zeros_like(carry_ref)

    x = x_ref[...]
    logits = jnp.dot(x, wr_ref[...], preferred_element_type=jnp.float32)
    m = jnp.max(logits, axis=1, keepdims=True)
    col = lax.broadcasted_iota(jnp.int32, logits.shape, 1)
    e_idx = jnp.min(jnp.where(logits == m, col, E), axis=1, keepdims=True)
    gate = 1.0 / jnp.sum(jnp.exp(logits - m), axis=1, keepdims=True)

    onehot = (col == e_idx).astype(jnp.float32)
    cum = onehot
    k = 1
    while k < R:
        shifted = jnp.concatenate([jnp.zeros((k, E), jnp.float32), cum[:-k]], 0)
        cum = cum + shifted
        k *= 2
    total = cum + carry_ref[...]
    carry_ref[...] = total[-1:, :]
    pos = jnp.sum((total - 1.0) * onehot, axis=1, keepdims=True)
    pos_i = pos.astype(jnp.int32)
    keep = pos_i < C
    slot_ref[...] = jnp.where(keep, e_idx * C + pos_i, E * C)
    xg_ref[...] = x * jnp.where(keep, gate, 0.0)


def _route(x2d, w_router, E, C, R):
    T, d = x2d.shape
    grid = (T // R,)
    slot2d, xg = pl.pallas_call(
        functools.partial(_route_body, E=E, C=C, R=R),
        grid=grid,
        in_specs=[
            pl.BlockSpec((R, d), lambda b: (b, 0)),
            pl.BlockSpec((d, E), lambda b: (0, 0)),
        ],
        out_specs=[
            pl.BlockSpec((R, 1), lambda b: (b, 0)),
            pl.BlockSpec((R, d), lambda b: (b, 0)),
        ],
        out_shape=[
            jax.ShapeDtypeStruct((T, 1), jnp.int32),
            jax.ShapeDtypeStruct((T, d), jnp.float32),
        ],
        scratch_shapes=[pltpu.VMEM((1, E), jnp.float32)],
    )(x2d, w_router)
    return slot2d.reshape(T), xg


def _dispatch(xg, slot3d, n_rows, CH):
    T, d = xg.shape
    tpw = T // _NW
    nch = tpw // CH
    mesh = plsc.VectorSubcoreMesh(core_axis_name="c", subcore_axis_name="s")

    @functools.partial(
        pl.kernel,
        out_type=jax.ShapeDtypeStruct((n_rows, d), jnp.float32),
        mesh=mesh,
        scratch_types=[
            pltpu.VMEM((nch, CH), jnp.int32),
            pltpu.VMEM((1, CH, d), jnp.float32),
            pltpu.SemaphoreType.DMA,
            pltpu.SemaphoreType.DMA,
            pltpu.SemaphoreType.DMA,
            pltpu.SemaphoreType.DMA,
        ],
    )
    def run(xg_hbm, slot_hbm, buf_hbm, idx_v, rows_v, ls0, ls1, ss0, ss1):
        del ls0, ls1, ss1
        wid = lax.axis_index("s") * _NC + lax.axis_index("c")
        base = wid * tpw
        pltpu.sync_copy(slot_hbm.at[wid], idx_v)
        for k in range(nch):
            pltpu.sync_copy(xg_hbm.at[pl.ds(base + k * CH, CH)], rows_v.at[0])
            pltpu.async_copy(rows_v.at[0], buf_hbm.at[idx_v.at[k]], ss0).wait()

    return run(xg, slot3d)


def _combine(o, slot3d, T, CH):
    n_rows, d = o.shape
    tpw = T // _NW
    nch = tpw // CH
    mesh = plsc.VectorSubcoreMesh(core_axis_name="c", subcore_axis_name="s")

    @functools.partial(
        pl.kernel,
        out_type=jax.ShapeDtypeStruct((T, d), jnp.float32),
        mesh=mesh,
        scratch_types=[
            pltpu.VMEM((nch, CH), jnp.int32),
            pltpu.VMEM((1, CH, d), jnp.float32),
            pltpu.SemaphoreType.DMA,
            pltpu.SemaphoreType.DMA,
            pltpu.SemaphoreType.DMA,
            pltpu.SemaphoreType.DMA,
        ],
    )
    def run(o_hbm, slot_hbm, y_hbm, idx_v, rows_v, gs0, gs1, ws0, ws1):
        del gs1, ws1
        wid = lax.axis_index("s") * _NC + lax.axis_index("c")
        base = wid * tpw
        pltpu.sync_copy(slot_hbm.at[wid], idx_v)
        for k in range(nch):
            pltpu.async_copy(o_hbm.at[idx_v.at[k]], rows_v.at[0], gs0).wait()
            pltpu.sync_copy(rows_v.at[0], y_hbm.at[pl.ds(base + k * CH, CH)])

    return run(o, slot3d)


def _ffn_body(buf_ref, w1_ref, w2_ref, o_ref, *, E):
    e = pl.program_id(0)

    @pl.when(e < E)
    def _compute():
        x = buf_ref[...].astype(jnp.bfloat16)
        h = jnp.maximum(
            jnp.dot(x, w1_ref[0].astype(jnp.bfloat16),
                    preferred_element_type=jnp.float32),
            0.0,
        )
        o_ref[...] = jnp.dot(h.astype(jnp.bfloat16),
                             w2_ref[0].astype(jnp.bfloat16),
                             preferred_element_type=jnp.float32)

    @pl.when(e == E)
    def _sentinel():
        o_ref[...] = jnp.zeros_like(o_ref)


def _ffn(buf, w1, w2, E, C):
    d = buf.shape[1]
    f = w1.shape[2]
    return pl.pallas_call(
        functools.partial(_ffn_body, E=E),
        grid=(E + 1,),
        in_specs=[
            pl.BlockSpec((C, d), lambda e: (e, 0)),
            pl.BlockSpec((1, d, f), lambda e: (jnp.minimum(e, E - 1), 0, 0)),
            pl.BlockSpec((1, f, d), lambda e: (jnp.minimum(e, E - 1), 0, 0)),
        ],
        out_specs=pl.BlockSpec((C, d), lambda e: (e, 0)),
        out_shape=jax.ShapeDtypeStruct(((E + 1) * C, d), jnp.float32),
    )(buf, w1, w2)


def kernel(inputs, w_router, w1, w2):
    Bq, Sq, d = inputs.shape
    T = Bq * Sq
    E = w1.shape[0]
    C = int(1.25 * T / E)

    x2d = inputs.reshape(T, d)
    CH = 128
    slot, xg = _route(x2d, w_router, E, C, R=1024)
    slot3d = slot.reshape(_NW, T // _NW // CH, CH)
    buf = _dispatch(xg, slot3d, n_rows=E * C + 8, CH=CH)
    o = _ffn(buf, w1, w2, E, C)
    y = _combine(o, slot3d, T, CH=CH)
    return y.reshape(Bq, Sq, d)
